# baseline (speedup 1.0000x reference)
import functools

import jax
import jax.numpy as jnp
from jax import lax
from jax.experimental import pallas as pl
from jax.experimental.pallas import tpu as pltpu
from jax.experimental.pallas import tpu_sc as plsc

_NC = 2
_NS = 16
_NW = _NC * _NS
_CH = 128


def _build_edge_pass(NP, D, K):
    R = NP // _NS
    NB = 8
    KH = 2
    T = K // NB
    NPIECE = R // _CH
    assert NPIECE + 2 <= NB and NPIECE <= NB
    mesh = plsc.VectorSubcoreMesh(
        core_axis_name="c", subcore_axis_name="s",
        num_cores=_NC, num_subcores=_NS)

    @functools.partial(
        pl.kernel,
        out_type=jax.ShapeDtypeStruct((_NC, NP, D), jnp.float32),
        mesh=mesh,
        compiler_params=pltpu.CompilerParams(use_tc_tiling_on_sc=False),
        scratch_types=[
            pltpu.VMEM((K, _CH), jnp.int32),
            pltpu.VMEM((K, _CH), jnp.int32),
            pltpu.VMEM((_CH, D), jnp.float32),
            pltpu.VMEM_SHARED((NP, D), jnp.float32),
            pltpu.VMEM_SHARED((NP, D), jnp.float32),
        ] + [pltpu.VMEM((_CH, D), jnp.float32) for _ in range(NB)]
          + [pltpu.SemaphoreType.DMA for _ in range(2 * NB)],
    )
    def edge_pass(z_hbm, src_hbm, dst_hbm, out_hbm,
                  src_v, dst_v, bounce, z_sp, acc, *rest):
        bufs = rest[:NB]
        gsems = rest[NB:2 * NB]
        ssems = rest[2 * NB:3 * NB]
        c = lax.axis_index("c")
        s = lax.axis_index("s")
        wid = c * _NS + s

        def piece(p):
            return pl.ds(s * R + p * _CH, _CH)

        pltpu.async_copy(src_hbm.at[wid], src_v, gsems[0])
        pltpu.async_copy(dst_hbm.at[wid], dst_v, gsems[1])
        for p in range(NPIECE):
            pltpu.async_copy(z_hbm.at[piece(p)], bufs[p], gsems[2 + p])

        zero16 = jnp.zeros((16,), jnp.float32)

        def zero_row(i, carry):
            for q in range(D // 16):
                bounce[i, pl.ds(q * 16, 16)] = zero16
            return carry

        lax.fori_loop(0, _CH, zero_row, 0)
        for p in range(NPIECE):
            pltpu.async_copy(bounce, acc.at[piece(p)], ssems[0])
        for p in range(NPIECE):
            pltpu.make_async_copy(z_hbm.at[piece(p)], bufs[p],
                                  gsems[2 + p]).wait()
            pltpu.async_copy(bufs[p], z_sp.at[piece(p)], ssems[1])
        for p in range(NPIECE):
            pltpu.make_async_copy(bounce, acc.at[piece(p)], ssems[0]).wait()
            pltpu.make_async_copy(bufs[p], z_sp.at[piece(p)], ssems[1]).wait()
        pltpu.make_async_copy(src_hbm.at[wid], src_v, gsems[0]).wait()
        pltpu.make_async_copy(dst_hbm.at[wid], dst_v, gsems[1]).wait()
        plsc.subcore_barrier()

        def gather_start(j, b, from_hbm):
            if from_hbm:
                pltpu.async_copy(z_hbm.at[src_v.at[j]], bufs[b], gsems[b])
            else:
                pltpu.async_copy(z_sp.at[src_v.at[j]], bufs[b], gsems[b])

        def gather_wait(j, b, from_hbm):
            if from_hbm:
                pltpu.make_async_copy(z_hbm.at[src_v.at[j]], bufs[b],
                                      gsems[b]).wait()
            else:
                pltpu.make_async_copy(z_sp.at[src_v.at[j]], bufs[b],
                                      gsems[b]).wait()

        for b in range(NB):
            gather_start(b, b, b < KH)

        def ring_step(t, carry):
            base = t * NB
            for b in range(NB):
                j = base + b
                gather_wait(j, b, b < KH)
                pltpu.async_copy(bufs[b], acc.at[dst_v.at[j]], ssems[b],
                                 add=True)
            for b in range(NB):
                j = base + b
                pltpu.make_async_copy(bufs[b], acc.at[dst_v.at[j]],
                                      ssems[b]).wait()

                @pl.when(j + NB < K)
                def _prefetch(b=b, j=j):
                    gather_start(j + NB, b, b < KH)

            return carry

        lax.fori_loop(0, T, ring_step, 0)
        plsc.subcore_barrier()

        for p in range(NPIECE):
            pltpu.async_copy(acc.at[piece(p)], bufs[p], gsems[p])
        for p in range(NPIECE):
            pltpu.make_async_copy(acc.at[piece(p)], bufs[p], gsems[p]).wait()
            pltpu.async_copy(bufs[p], out_hbm.at[c, piece(p)], ssems[p])
        for p in range(NPIECE):
            pltpu.make_async_copy(bufs[p], out_hbm.at[c, piece(p)],
                                  ssems[p]).wait()

    return edge_pass


def _build_deg_pass(NP, K):
    DW = 8
    R = NP // _NS
    mesh = plsc.VectorSubcoreMesh(
        core_axis_name="c", subcore_axis_name="s",
        num_cores=_NC, num_subcores=_NS)

    @functools.partial(
        pl.kernel,
        out_type=(jax.ShapeDtypeStruct((_NC, NP, DW), jnp.float32),
                  jax.ShapeDtypeStruct((_NC, NP, DW), jnp.float32)),
        mesh=mesh,
        compiler_params=pltpu.CompilerParams(use_tc_tiling_on_sc=False),
        scratch_types=[
            pltpu.VMEM((K, _CH), jnp.int32),
            pltpu.VMEM((K, _CH), jnp.int32),
            pltpu.VMEM((_CH, DW), jnp.float32),
            pltpu.VMEM((R, DW), jnp.float32),
            pltpu.VMEM_SHARED((NP, DW), jnp.float32),
            pltpu.VMEM_SHARED((NP, DW), jnp.float32),
            pltpu.SemaphoreType.DMA,
            pltpu.SemaphoreType.DMA,
        ],
    )
    def deg_pass(ones_hbm, zeros_hbm, src_hbm, dst_hbm, outS_hbm, outD_hbm,
                 src_v, dst_v, ones, bounce, accS, accD, semS, semD):
        c = lax.axis_index("c")
        s = lax.axis_index("s")
        wid = c * _NS + s

        pltpu.async_copy(src_hbm.at[wid], src_v, semS)
        pltpu.async_copy(dst_hbm.at[wid], dst_v, semD)
        pltpu.sync_copy(ones_hbm, ones)
        pltpu.sync_copy(zeros_hbm, bounce)
        pltpu.sync_copy(bounce, accS.at[pl.ds(s * R, R)])
        pltpu.sync_copy(bounce, accD.at[pl.ds(s * R, R)])
        pltpu.make_async_copy(src_hbm.at[wid], src_v, semS).wait()
        pltpu.make_async_copy(dst_hbm.at[wid], dst_v, semD).wait()
        plsc.subcore_barrier()

        def chunk(t, carry):
            pltpu.async_copy(ones, accS.at[src_v.at[t]], semS, add=True)
            pltpu.async_copy(ones, accD.at[dst_v.at[t]], semD, add=True)

            @pl.when(t > 0)
            def _drain():
                pltpu.make_async_copy(ones, accS.at[src_v.at[t - 1]],
                                      semS).wait()
                pltpu.make_async_copy(ones, accD.at[dst_v.at[t - 1]],
                                      semD).wait()

            return carry

        lax.fori_loop(0, K, chunk, 0)
        pltpu.make_async_copy(ones, accS.at[src_v.at[K - 1]], semS).wait()
        pltpu.make_async_copy(ones, accD.at[dst_v.at[K - 1]], semD).wait()
        plsc.subcore_barrier()

        pltpu.sync_copy(accS.at[pl.ds(s * R, R)], bounce)
        pltpu.sync_copy(bounce, outS_hbm.at[c, pl.ds(s * R, R)])
        pltpu.sync_copy(accD.at[pl.ds(s * R, R)], bounce)
        pltpu.sync_copy(bounce, outD_hbm.at[c, pl.ds(s * R, R)])

    return deg_pass


def kernel(feats, edge_index, W_in, b_in, W_hid, b_hid, W_out, b_out):
    N, d_in = feats.shape
    E = edge_index.shape[1]
    n_layers, d_h, _ = W_hid.shape
    d_out = W_out.shape[1]

    NP = -(-(N + 1) // 256) * 256
    EP = -(-E // (_NW * 2 * _CH)) * (_NW * 2 * _CH)
    K = EP // (_NW * _CH)

    pad = EP - E
    src = jnp.concatenate([edge_index[0], jnp.full((pad,), N, jnp.int32)])
    dst = jnp.concatenate([edge_index[1], jnp.full((pad,), N, jnp.int32)])
    src3 = src.reshape(_NW, K, _CH)
    dst3 = dst.reshape(_NW, K, _CH)

    PK = 128 // d_h
    NG = NP // PK
    n_convs = n_layers + 1
    n_split = d_out // d_h

    def packed(a):
        return a.reshape(a.shape[:-2] + (NG, PK * d_h))

    def unpacked(a):
        return a.reshape(NP, d_h)

    eye4 = jnp.eye(PK, dtype=jnp.float32)
    b_in4 = jnp.tile(b_in.reshape(1, d_h), (1, PK))
    b_hid4 = jnp.tile(b_hid.reshape(n_layers, 1, d_h), (1, 1, PK))
    b_out2 = b_out.reshape(1, d_out)
    W_hid4 = jnp.stack([jnp.kron(eye4, W_hid[i]) for i in range(n_layers)])
    Wout4 = jnp.concatenate(
        [jnp.kron(eye4, W_out[i * d_h:(i + 1) * d_h]) for i in range(n_convs)],
        axis=0)

    deg_pass = _build_deg_pass(NP, K)
    edge32 = _build_edge_pass(NP, d_h, K)

    ones_c = jnp.ones((_CH, 8), jnp.float32)
    zeros_c = jnp.zeros((NP // _NS, 8), jnp.float32)
    degS, degD = deg_pass(ones_c, zeros_c, src3, dst3)
    degS = degS.reshape(_NC, NG, PK * 8)
    degD = degD.reshape(_NC, NG, PK * 8)

    NREAL = N // PK
    feats_p = feats.reshape(NREAL, PK * d_in)
    W_in4 = jnp.kron(eye4, W_in)

    def tc_proj(f_ref, w_ref, z_ref):
        zp = jnp.dot(f_ref[...], w_ref[...], preferred_element_type=jnp.float32)
        tail = jnp.zeros((NG - NREAL, PK * d_h), jnp.float32)
        z_ref[...] = jnp.concatenate([zp, tail], axis=0)

    z_raw = pl.pallas_call(
        tc_proj,
        out_shape=jax.ShapeDtypeStruct((NG, PK * d_h), jnp.float32),
    )(feats_p, W_in4)

    GB = 8
    NGB = NG // GB

    def tc_norms(dS_ref, dD_ref, zr_ref, ns_ref, nd_ref, z_ref):
        dS = dS_ref[0] + dS_ref[1]
        dD = dD_ref[0] + dD_ref[1]

        def spread(d):
            cols = [jnp.broadcast_to(d[:, 8 * k:8 * k + 1], (d.shape[0], d_h))
                    for k in range(PK)]
            return jnp.concatenate(cols, axis=1)

        ns = lax.rsqrt(jnp.maximum(spread(dS), 1.0))
        nd = lax.rsqrt(jnp.maximum(spread(dD), 1.0))
        ns_ref[...] = ns
        nd_ref[...] = nd
        z_ref[...] = zr_ref[...] * ns

    ns_arr, nd_arr, z = pl.pallas_call(
        tc_norms,
        grid=(GB,),
        in_specs=[pl.BlockSpec((_NC, NGB, PK * 8), lambda i: (0, i, 0)),
                  pl.BlockSpec((_NC, NGB, PK * 8), lambda i: (0, i, 0)),
                  pl.BlockSpec((NGB, PK * d_h), lambda i: (i, 0))],
        out_specs=(pl.BlockSpec((NGB, PK * d_h), lambda i: (i, 0)),
                   pl.BlockSpec((NGB, PK * d_h), lambda i: (i, 0)),
                   pl.BlockSpec((NGB, PK * d_h), lambda i: (i, 0))),
        out_shape=(jax.ShapeDtypeStruct((NG, PK * d_h), jnp.float32),
                   jax.ShapeDtypeStruct((NG, PK * d_h), jnp.float32),
                   jax.ShapeDtypeStruct((NG, PK * d_h), jnp.float32)),
    )(degS, degD, z_raw)

    def tc_layer(p_ref, nd_ref, ns_ref, b_ref, w_ref, h_ref, z_ref):
        agg = p_ref[0] + p_ref[1]
        h = jnp.maximum(agg * nd_ref[...] + b_ref[...], 0.0)
        h_ref[...] = h
        z_ref[...] = jnp.dot(h, w_ref[...],
                             preferred_element_type=jnp.float32) * ns_ref[...]

    tc_layer_call = pl.pallas_call(
        tc_layer,
        grid=(GB,),
        in_specs=[pl.BlockSpec((_NC, NGB, PK * d_h), lambda i: (0, i, 0)),
                  pl.BlockSpec((NGB, PK * d_h), lambda i: (i, 0)),
                  pl.BlockSpec((NGB, PK * d_h), lambda i: (i, 0)),
                  pl.BlockSpec((1, PK * d_h), lambda i: (0, 0)),
                  pl.BlockSpec((PK * d_h, PK * d_h), lambda i: (0, 0))],
        out_specs=(pl.BlockSpec((NGB, PK * d_h), lambda i: (i, 0)),
                   pl.BlockSpec((NGB, PK * d_h), lambda i: (i, 0))),
        out_shape=(jax.ShapeDtypeStruct((NG, PK * d_h), jnp.float32),
                   jax.ShapeDtypeStruct((NG, PK * d_h), jnp.float32)),
    )

    hs = []
    for i in range(n_layers):
        part = packed(edge32(unpacked(z), src3, dst3))
        bias = b_in4 if i == 0 else b_hid4[i - 1]
        h, z = tc_layer_call(part, nd_arr, ns_arr, bias, W_hid4[i])
        hs.append(h)
    part_last = packed(edge32(unpacked(z), src3, dst3))

    def tc_jk(p_ref, nd_ref, b_ref, *rest):
        h_refs = rest[:n_layers]
        wout_ref = rest[n_layers]
        out_refs = rest[n_layers + 1:]
        agg = p_ref[0] + p_ref[1]
        h_last = jnp.maximum(agg * nd_ref[...] + b_ref[...], 0.0)
        hcat = jnp.concatenate([r[...] for r in h_refs] + [h_last], axis=1)
        P = jnp.dot(hcat, wout_ref[...], preferred_element_type=jnp.float32)
        for i, o_ref in enumerate(out_refs):
            o_ref[...] = jnp.concatenate(
                [P[:, k * d_out + i * d_h: k * d_out + (i + 1) * d_h]
                 for k in range(PK)], axis=1)

    Ps = pl.pallas_call(
        tc_jk,
        grid=(GB,),
        in_specs=[pl.BlockSpec((_NC, NGB, PK * d_h), lambda i: (0, i, 0)),
                  pl.BlockSpec((NGB, PK * d_h), lambda i: (i, 0)),
                  pl.BlockSpec((1, PK * d_h), lambda i: (0, 0))]
                 + [pl.BlockSpec((NGB, PK * d_h), lambda i: (i, 0))
                    for _ in range(n_layers)]
                 + [pl.BlockSpec((n_convs * PK * d_h, PK * d_out),
                                 lambda i: (0, 0))],
        out_specs=tuple(pl.BlockSpec((NGB, PK * d_h), lambda i: (i, 0))
                        for _ in range(n_split)),
        out_shape=tuple(jax.ShapeDtypeStruct((NG, PK * d_h), jnp.float32)
                        for _ in range(n_split)),
    )(part_last, nd_arr, b_hid4[n_layers - 1], *hs, Wout4)

    partFs = [packed(edge32(unpacked(P_i), src3, dst3)) for P_i in Ps]

    b_out4 = jnp.tile(b_out.reshape(1, d_out), (1, PK))

    def tc_final(*refs):
        p_refs, b_ref, y_ref = refs[:n_split], refs[n_split], refs[n_split + 1]
        fs = [p_ref[0] + p_ref[1] for p_ref in p_refs]
        cols = []
        for k in range(PK):
            for f in fs:
                cols.append(f[:, k * d_h:(k + 1) * d_h])
        y_ref[...] = jnp.concatenate(cols, axis=1) + b_ref[...]

    YB = NG // GB
    y_pk = pl.pallas_call(
        tc_final,
        grid=(GB,),
        in_specs=[pl.BlockSpec((_NC, YB, PK * d_h), lambda i: (0, i, 0))
                  for _ in range(n_split)]
                 + [pl.BlockSpec((1, PK * d_out), lambda i: (0, 0))],
        out_specs=pl.BlockSpec((YB, PK * d_out), lambda i: (i, 0)),
        out_shape=jax.ShapeDtypeStruct((NG, PK * d_out), jnp.float32),
    )(*partFs, b_out4)
    return y_pk.reshape(NP, d_out)[:N]


# speedup vs baseline: 1.0301x; 1.0301x over previous
import functools

import jax
import jax.numpy as jnp
from jax import lax
from jax.experimental import pallas as pl
from jax.experimental.pallas import tpu as pltpu
from jax.experimental.pallas import tpu_sc as plsc

_NC = 2
_NS = 16
_NW = _NC * _NS
_CH = 128


def _build_edge_pass(NP, D, K):
    R = NP // _NS
    NB = 8
    KH = 0
    T = K // NB
    NPIECE = R // _CH
    assert NPIECE + 2 <= NB and NPIECE <= NB
    mesh = plsc.VectorSubcoreMesh(
        core_axis_name="c", subcore_axis_name="s",
        num_cores=_NC, num_subcores=_NS)

    @functools.partial(
        pl.kernel,
        out_type=jax.ShapeDtypeStruct((_NC, NP, D), jnp.float32),
        mesh=mesh,
        compiler_params=pltpu.CompilerParams(use_tc_tiling_on_sc=False),
        scratch_types=[
            pltpu.VMEM((K, _CH), jnp.int32),
            pltpu.VMEM((K, _CH), jnp.int32),
            pltpu.VMEM((_CH, D), jnp.float32),
            pltpu.VMEM_SHARED((NP, D), jnp.float32),
            pltpu.VMEM_SHARED((NP, D), jnp.float32),
        ] + [pltpu.VMEM((_CH, D), jnp.float32) for _ in range(NB)]
          + [pltpu.SemaphoreType.DMA for _ in range(2 * NB)],
    )
    def edge_pass(z_hbm, src_hbm, dst_hbm, out_hbm,
                  src_v, dst_v, bounce, z_sp, acc, *rest):
        bufs = rest[:NB]
        gsems = rest[NB:2 * NB]
        ssems = rest[2 * NB:3 * NB]
        c = lax.axis_index("c")
        s = lax.axis_index("s")
        wid = c * _NS + s

        def piece(p):
            return pl.ds(s * R + p * _CH, _CH)

        pltpu.async_copy(src_hbm.at[wid], src_v, gsems[0])
        pltpu.async_copy(dst_hbm.at[wid], dst_v, gsems[1])
        zrows = pl.ds(s * R, R)
        pltpu.async_copy(z_hbm.at[zrows], z_sp.at[zrows], gsems[2])

        zero16 = jnp.zeros((16,), jnp.float32)

        def zero_row(i, carry):
            for q in range(D // 16):
                bounce[i, pl.ds(q * 16, 16)] = zero16
            return carry

        lax.fori_loop(0, _CH, zero_row, 0)
        for p in range(NPIECE):
            pltpu.async_copy(bounce, acc.at[piece(p)], ssems[0])
        for p in range(NPIECE):
            pltpu.make_async_copy(bounce, acc.at[piece(p)], ssems[0]).wait()
        pltpu.make_async_copy(z_hbm.at[zrows], z_sp.at[zrows], gsems[2]).wait()
        pltpu.make_async_copy(src_hbm.at[wid], src_v, gsems[0]).wait()
        pltpu.make_async_copy(dst_hbm.at[wid], dst_v, gsems[1]).wait()
        plsc.subcore_barrier()

        def gather_start(j, b, from_hbm):
            src = z_hbm if from_hbm else z_sp
            pltpu.async_copy(src.at[src_v.at[j]], bufs[b], gsems[b])

        def gather_wait(j, b, from_hbm):
            src = z_hbm if from_hbm else z_sp
            pltpu.make_async_copy(src.at[src_v.at[j]], bufs[b],
                                  gsems[b]).wait()

        for b in range(NB):
            gather_start(b, b, b < KH)

        def ring_step(t, carry):
            base = t * NB
            for b in range(NB):
                j = base + b
                gather_wait(j, b, b < KH)
                pltpu.async_copy(bufs[b], acc.at[dst_v.at[j]], ssems[b],
                                 add=True)
            for b in range(NB):
                j = base + b
                pltpu.make_async_copy(bufs[b], acc.at[dst_v.at[j]],
                                      ssems[b]).wait()

                @pl.when(j + NB < K)
                def _prefetch(b=b, j=j):
                    gather_start(j + NB, b, b < KH)

            return carry

        lax.fori_loop(0, T, ring_step, 0)
        plsc.subcore_barrier()

        pltpu.sync_copy(acc.at[zrows], out_hbm.at[c, zrows])

    return edge_pass


def _build_deg_pass(NP, K):
    DW = 8
    R = NP // _NS
    mesh = plsc.VectorSubcoreMesh(
        core_axis_name="c", subcore_axis_name="s",
        num_cores=_NC, num_subcores=_NS)

    @functools.partial(
        pl.kernel,
        out_type=(jax.ShapeDtypeStruct((_NC, NP, DW), jnp.float32),
                  jax.ShapeDtypeStruct((_NC, NP, DW), jnp.float32)),
        mesh=mesh,
        compiler_params=pltpu.CompilerParams(use_tc_tiling_on_sc=False),
        scratch_types=[
            pltpu.VMEM((K, _CH), jnp.int32),
            pltpu.VMEM((K, _CH), jnp.int32),
            pltpu.VMEM((_CH, DW), jnp.float32),
            pltpu.VMEM((R, DW), jnp.float32),
            pltpu.VMEM_SHARED((NP, DW), jnp.float32),
            pltpu.VMEM_SHARED((NP, DW), jnp.float32),
            pltpu.SemaphoreType.DMA,
            pltpu.SemaphoreType.DMA,
        ],
    )
    def deg_pass(ones_hbm, zeros_hbm, src_hbm, dst_hbm, outS_hbm, outD_hbm,
                 src_v, dst_v, ones, bounce, accS, accD, semS, semD):
        c = lax.axis_index("c")
        s = lax.axis_index("s")
        wid = c * _NS + s

        pltpu.async_copy(src_hbm.at[wid], src_v, semS)
        pltpu.async_copy(dst_hbm.at[wid], dst_v, semD)
        pltpu.sync_copy(ones_hbm, ones)
        pltpu.sync_copy(zeros_hbm, bounce)
        pltpu.sync_copy(bounce, accS.at[pl.ds(s * R, R)])
        pltpu.sync_copy(bounce, accD.at[pl.ds(s * R, R)])
        pltpu.make_async_copy(src_hbm.at[wid], src_v, semS).wait()
        pltpu.make_async_copy(dst_hbm.at[wid], dst_v, semD).wait()
        plsc.subcore_barrier()

        def chunk(t, carry):
            pltpu.async_copy(ones, accS.at[src_v.at[t]], semS, add=True)
            pltpu.async_copy(ones, accD.at[dst_v.at[t]], semD, add=True)

            @pl.when(t > 0)
            def _drain():
                pltpu.make_async_copy(ones, accS.at[src_v.at[t - 1]],
                                      semS).wait()
                pltpu.make_async_copy(ones, accD.at[dst_v.at[t - 1]],
                                      semD).wait()

            return carry

        lax.fori_loop(0, K, chunk, 0)
        pltpu.make_async_copy(ones, accS.at[src_v.at[K - 1]], semS).wait()
        pltpu.make_async_copy(ones, accD.at[dst_v.at[K - 1]], semD).wait()
        plsc.subcore_barrier()

        pltpu.sync_copy(accS.at[pl.ds(s * R, R)], bounce)
        pltpu.sync_copy(bounce, outS_hbm.at[c, pl.ds(s * R, R)])
        pltpu.sync_copy(accD.at[pl.ds(s * R, R)], bounce)
        pltpu.sync_copy(bounce, outD_hbm.at[c, pl.ds(s * R, R)])

    return deg_pass


def kernel(feats, edge_index, W_in, b_in, W_hid, b_hid, W_out, b_out):
    N, d_in = feats.shape
    E = edge_index.shape[1]
    n_layers, d_h, _ = W_hid.shape
    d_out = W_out.shape[1]

    NP = -(-(N + 1) // 256) * 256
    EP = -(-E // (_NW * 2 * _CH)) * (_NW * 2 * _CH)
    K = EP // (_NW * _CH)

    pad = EP - E
    src = jnp.concatenate([edge_index[0], jnp.full((pad,), N, jnp.int32)])
    dst = jnp.concatenate([edge_index[1], jnp.full((pad,), N, jnp.int32)])
    src3 = src.reshape(_NW, K, _CH)
    dst3 = dst.reshape(_NW, K, _CH)

    PK = 128 // d_h
    NG = NP // PK
    n_convs = n_layers + 1
    n_split = d_out // d_h

    def packed(a):
        return a.reshape(a.shape[:-2] + (NG, PK * d_h))

    def unpacked(a):
        return a.reshape(NP, d_h)

    eye4 = jnp.eye(PK, dtype=jnp.float32)
    b_in4 = jnp.tile(b_in.reshape(1, d_h), (1, PK))
    b_hid4 = jnp.tile(b_hid.reshape(n_layers, 1, d_h), (1, 1, PK))
    b_out2 = b_out.reshape(1, d_out)
    W_hid4 = jnp.stack([jnp.kron(eye4, W_hid[i]) for i in range(n_layers)])
    Wout4 = jnp.concatenate(
        [jnp.kron(eye4, W_out[i * d_h:(i + 1) * d_h]) for i in range(n_convs)],
        axis=0)

    deg_pass = _build_deg_pass(NP, K)
    edge32 = _build_edge_pass(NP, d_h, K)

    ones_c = jnp.ones((_CH, 8), jnp.float32)
    zeros_c = jnp.zeros((NP // _NS, 8), jnp.float32)
    degS, degD = deg_pass(ones_c, zeros_c, src3, dst3)
    degS = degS.reshape(_NC, NG, PK * 8)
    degD = degD.reshape(_NC, NG, PK * 8)

    NREAL = N // PK
    feats_p = feats.reshape(NREAL, PK * d_in)
    W_in4 = jnp.kron(eye4, W_in)

    def tc_proj(f_ref, w_ref, z_ref):
        zp = jnp.dot(f_ref[...], w_ref[...], preferred_element_type=jnp.float32)
        tail = jnp.zeros((NG - NREAL, PK * d_h), jnp.float32)
        z_ref[...] = jnp.concatenate([zp, tail], axis=0)

    z_raw = pl.pallas_call(
        tc_proj,
        out_shape=jax.ShapeDtypeStruct((NG, PK * d_h), jnp.float32),
    )(feats_p, W_in4)

    GB = 8
    NGB = NG // GB

    def tc_norms(dS_ref, dD_ref, zr_ref, ns_ref, nd_ref, z_ref):
        dS = dS_ref[0] + dS_ref[1]
        dD = dD_ref[0] + dD_ref[1]

        def spread(d):
            cols = [jnp.broadcast_to(d[:, 8 * k:8 * k + 1], (d.shape[0], d_h))
                    for k in range(PK)]
            return jnp.concatenate(cols, axis=1)

        ns = lax.rsqrt(jnp.maximum(spread(dS), 1.0))
        nd = lax.rsqrt(jnp.maximum(spread(dD), 1.0))
        ns_ref[...] = ns
        nd_ref[...] = nd
        z_ref[...] = zr_ref[...] * ns

    ns_arr, nd_arr, z = pl.pallas_call(
        tc_norms,
        grid=(GB,),
        in_specs=[pl.BlockSpec((_NC, NGB, PK * 8), lambda i: (0, i, 0)),
                  pl.BlockSpec((_NC, NGB, PK * 8), lambda i: (0, i, 0)),
                  pl.BlockSpec((NGB, PK * d_h), lambda i: (i, 0))],
        out_specs=(pl.BlockSpec((NGB, PK * d_h), lambda i: (i, 0)),
                   pl.BlockSpec((NGB, PK * d_h), lambda i: (i, 0)),
                   pl.BlockSpec((NGB, PK * d_h), lambda i: (i, 0))),
        out_shape=(jax.ShapeDtypeStruct((NG, PK * d_h), jnp.float32),
                   jax.ShapeDtypeStruct((NG, PK * d_h), jnp.float32),
                   jax.ShapeDtypeStruct((NG, PK * d_h), jnp.float32)),
    )(degS, degD, z_raw)

    def tc_layer(p_ref, nd_ref, ns_ref, b_ref, w_ref, h_ref, z_ref):
        agg = p_ref[0] + p_ref[1]
        h = jnp.maximum(agg * nd_ref[...] + b_ref[...], 0.0)
        h_ref[...] = h
        z_ref[...] = jnp.dot(h, w_ref[...],
                             preferred_element_type=jnp.float32) * ns_ref[...]

    tc_layer_call = pl.pallas_call(
        tc_layer,
        grid=(GB,),
        in_specs=[pl.BlockSpec((_NC, NGB, PK * d_h), lambda i: (0, i, 0)),
                  pl.BlockSpec((NGB, PK * d_h), lambda i: (i, 0)),
                  pl.BlockSpec((NGB, PK * d_h), lambda i: (i, 0)),
                  pl.BlockSpec((1, PK * d_h), lambda i: (0, 0)),
                  pl.BlockSpec((PK * d_h, PK * d_h), lambda i: (0, 0))],
        out_specs=(pl.BlockSpec((NGB, PK * d_h), lambda i: (i, 0)),
                   pl.BlockSpec((NGB, PK * d_h), lambda i: (i, 0))),
        out_shape=(jax.ShapeDtypeStruct((NG, PK * d_h), jnp.float32),
                   jax.ShapeDtypeStruct((NG, PK * d_h), jnp.float32)),
    )

    hs = []
    for i in range(n_layers):
        part = packed(edge32(unpacked(z), src3, dst3))
        bias = b_in4 if i == 0 else b_hid4[i - 1]
        h, z = tc_layer_call(part, nd_arr, ns_arr, bias, W_hid4[i])
        hs.append(h)
    part_last = packed(edge32(unpacked(z), src3, dst3))

    def tc_jk(p_ref, nd_ref, b_ref, *rest):
        h_refs = rest[:n_layers]
        wout_ref = rest[n_layers]
        out_refs = rest[n_layers + 1:]
        agg = p_ref[0] + p_ref[1]
        h_last = jnp.maximum(agg * nd_ref[...] + b_ref[...], 0.0)
        hcat = jnp.concatenate([r[...] for r in h_refs] + [h_last], axis=1)
        P = jnp.dot(hcat, wout_ref[...], preferred_element_type=jnp.float32)
        for i, o_ref in enumerate(out_refs):
            o_ref[...] = jnp.concatenate(
                [P[:, k * d_out + i * d_h: k * d_out + (i + 1) * d_h]
                 for k in range(PK)], axis=1)

    Ps = pl.pallas_call(
        tc_jk,
        grid=(GB,),
        in_specs=[pl.BlockSpec((_NC, NGB, PK * d_h), lambda i: (0, i, 0)),
                  pl.BlockSpec((NGB, PK * d_h), lambda i: (i, 0)),
                  pl.BlockSpec((1, PK * d_h), lambda i: (0, 0))]
                 + [pl.BlockSpec((NGB, PK * d_h), lambda i: (i, 0))
                    for _ in range(n_layers)]
                 + [pl.BlockSpec((n_convs * PK * d_h, PK * d_out),
                                 lambda i: (0, 0))],
        out_specs=tuple(pl.BlockSpec((NGB, PK * d_h), lambda i: (i, 0))
                        for _ in range(n_split)),
        out_shape=tuple(jax.ShapeDtypeStruct((NG, PK * d_h), jnp.float32)
                        for _ in range(n_split)),
    )(part_last, nd_arr, b_hid4[n_layers - 1], *hs, Wout4)

    partFs = [packed(edge32(unpacked(P_i), src3, dst3)) for P_i in Ps]

    b_out4 = jnp.tile(b_out.reshape(1, d_out), (1, PK))

    def tc_final(*refs):
        p_refs, b_ref, y_ref = refs[:n_split], refs[n_split], refs[n_split + 1]
        fs = [p_ref[0] + p_ref[1] for p_ref in p_refs]
        cols = []
        for k in range(PK):
            for f in fs:
                cols.append(f[:, k * d_h:(k + 1) * d_h])
        y_ref[...] = jnp.concatenate(cols, axis=1) + b_ref[...]

    YB = NG // GB
    y_pk = pl.pallas_call(
        tc_final,
        grid=(GB,),
        in_specs=[pl.BlockSpec((_NC, YB, PK * d_h), lambda i: (0, i, 0))
                  for _ in range(n_split)]
                 + [pl.BlockSpec((1, PK * d_out), lambda i: (0, 0))],
        out_specs=pl.BlockSpec((YB, PK * d_out), lambda i: (i, 0)),
        out_shape=jax.ShapeDtypeStruct((NG, PK * d_out), jnp.float32),
    )(*partFs, b_out4)
    return y_pk.reshape(NP, d_out)[:N]


# speedup vs baseline: 1.0315x; 1.0013x over previous
import functools

import jax
import jax.numpy as jnp
from jax import lax
from jax.experimental import pallas as pl
from jax.experimental.pallas import tpu as pltpu
from jax.experimental.pallas import tpu_sc as plsc

_NC = 2
_NS = 16
_NW = _NC * _NS
_CH = 128


def _build_edge_pass(NP, D, K):
    R = NP // _NS
    NB = 8
    KH = 0
    T = K // NB
    NPIECE = R // _CH
    assert NPIECE + 2 <= NB and NPIECE <= NB
    mesh = plsc.VectorSubcoreMesh(
        core_axis_name="c", subcore_axis_name="s",
        num_cores=_NC, num_subcores=_NS)

    @functools.partial(
        pl.kernel,
        out_type=jax.ShapeDtypeStruct((_NC, NP, D), jnp.float32),
        mesh=mesh,
        compiler_params=pltpu.CompilerParams(use_tc_tiling_on_sc=False),
        scratch_types=[
            pltpu.VMEM((K, _CH), jnp.int32),
            pltpu.VMEM((K, _CH), jnp.int32),
            pltpu.VMEM((_CH, D), jnp.float32),
            pltpu.VMEM_SHARED((NP, D), jnp.float32),
            pltpu.VMEM_SHARED((NP, D), jnp.float32),
        ] + [pltpu.VMEM((_CH, D), jnp.float32) for _ in range(NB)]
          + [pltpu.SemaphoreType.DMA for _ in range(2 * NB)],
    )
    def edge_pass(z_hbm, src_hbm, dst_hbm, out_hbm,
                  src_v, dst_v, bounce, z_sp, acc, *rest):
        bufs = rest[:NB]
        gsems = rest[NB:2 * NB]
        ssems = rest[2 * NB:3 * NB]
        c = lax.axis_index("c")
        s = lax.axis_index("s")
        wid = c * _NS + s

        def piece(p):
            return pl.ds(s * R + p * _CH, _CH)

        pltpu.async_copy(src_hbm.at[wid], src_v, gsems[0])
        pltpu.async_copy(dst_hbm.at[wid], dst_v, gsems[1])
        zrows = pl.ds(s * R, R)
        pltpu.async_copy(z_hbm.at[zrows], z_sp.at[zrows], gsems[2])

        zero16 = jnp.zeros((16,), jnp.float32)

        def zero_row(i, carry):
            for q in range(D // 16):
                bounce[i, pl.ds(q * 16, 16)] = zero16
            return carry

        lax.fori_loop(0, _CH, zero_row, 0)
        for p in range(NPIECE):
            pltpu.async_copy(bounce, acc.at[piece(p)], ssems[0])
        for p in range(NPIECE):
            pltpu.make_async_copy(bounce, acc.at[piece(p)], ssems[0]).wait()
        pltpu.make_async_copy(z_hbm.at[zrows], z_sp.at[zrows], gsems[2]).wait()
        pltpu.make_async_copy(src_hbm.at[wid], src_v, gsems[0]).wait()
        pltpu.make_async_copy(dst_hbm.at[wid], dst_v, gsems[1]).wait()
        plsc.subcore_barrier()

        def gather_start(j, b, from_hbm):
            src = z_hbm if from_hbm else z_sp
            pltpu.async_copy(src.at[src_v.at[j]], bufs[b], gsems[b])

        def gather_wait(j, b, from_hbm):
            src = z_hbm if from_hbm else z_sp
            pltpu.make_async_copy(src.at[src_v.at[j]], bufs[b],
                                  gsems[b]).wait()

        for b in range(NB):
            gather_start(b, b, b < KH)

        def ring_step(t, carry):
            base = t * NB
            for b in range(NB):
                j = base + b
                gather_wait(j, b, b < KH)
                pltpu.async_copy(bufs[b], acc.at[dst_v.at[j]], ssems[b],
                                 add=True)
            for b in range(NB):
                j = base + b
                pltpu.make_async_copy(bufs[b], acc.at[dst_v.at[j]],
                                      ssems[b]).wait()

                @pl.when(j + NB < K)
                def _prefetch(b=b, j=j):
                    gather_start(j + NB, b, b < KH)

            return carry

        lax.fori_loop(0, T, ring_step, 0)
        plsc.subcore_barrier()

        pltpu.sync_copy(acc.at[zrows], out_hbm.at[c, zrows])

    return edge_pass


def _build_deg_pass(NP, K):
    DW = 8
    R = NP // _NS
    mesh = plsc.VectorSubcoreMesh(
        core_axis_name="c", subcore_axis_name="s",
        num_cores=_NC, num_subcores=_NS)

    @functools.partial(
        pl.kernel,
        out_type=(jax.ShapeDtypeStruct((_NC, NP, DW), jnp.float32),
                  jax.ShapeDtypeStruct((_NC, NP, DW), jnp.float32)),
        mesh=mesh,
        compiler_params=pltpu.CompilerParams(use_tc_tiling_on_sc=False),
        scratch_types=[
            pltpu.VMEM((K, _CH), jnp.int32),
            pltpu.VMEM((K, _CH), jnp.int32),
            pltpu.VMEM((_CH, DW), jnp.float32),
            pltpu.VMEM((R, DW), jnp.float32),
            pltpu.VMEM_SHARED((NP, DW), jnp.float32),
            pltpu.VMEM_SHARED((NP, DW), jnp.float32),
            pltpu.SemaphoreType.DMA,
            pltpu.SemaphoreType.DMA,
        ],
    )
    def deg_pass(ones_hbm, zeros_hbm, src_hbm, dst_hbm, outS_hbm, outD_hbm,
                 src_v, dst_v, ones, bounce, accS, accD, semS, semD):
        c = lax.axis_index("c")
        s = lax.axis_index("s")
        wid = c * _NS + s

        pltpu.async_copy(src_hbm.at[wid], src_v, semS)
        pltpu.async_copy(dst_hbm.at[wid], dst_v, semD)
        pltpu.sync_copy(ones_hbm, ones)
        pltpu.sync_copy(zeros_hbm, bounce)
        pltpu.sync_copy(bounce, accS.at[pl.ds(s * R, R)])
        pltpu.sync_copy(bounce, accD.at[pl.ds(s * R, R)])
        pltpu.make_async_copy(src_hbm.at[wid], src_v, semS).wait()
        pltpu.make_async_copy(dst_hbm.at[wid], dst_v, semD).wait()
        plsc.subcore_barrier()

        def chunk(t, carry):
            pltpu.async_copy(ones, accS.at[src_v.at[t]], semS, add=True)
            pltpu.async_copy(ones, accD.at[dst_v.at[t]], semD, add=True)

            @pl.when(t > 0)
            def _drain():
                pltpu.make_async_copy(ones, accS.at[src_v.at[t - 1]],
                                      semS).wait()
                pltpu.make_async_copy(ones, accD.at[dst_v.at[t - 1]],
                                      semD).wait()

            return carry

        lax.fori_loop(0, K, chunk, 0)
        pltpu.make_async_copy(ones, accS.at[src_v.at[K - 1]], semS).wait()
        pltpu.make_async_copy(ones, accD.at[dst_v.at[K - 1]], semD).wait()
        plsc.subcore_barrier()

        pltpu.sync_copy(accS.at[pl.ds(s * R, R)], bounce)
        pltpu.sync_copy(bounce, outS_hbm.at[c, pl.ds(s * R, R)])
        pltpu.sync_copy(accD.at[pl.ds(s * R, R)], bounce)
        pltpu.sync_copy(bounce, outD_hbm.at[c, pl.ds(s * R, R)])

    return deg_pass


def kernel(feats, edge_index, W_in, b_in, W_hid, b_hid, W_out, b_out):
    N, d_in = feats.shape
    E = edge_index.shape[1]
    n_layers, d_h, _ = W_hid.shape
    d_out = W_out.shape[1]

    NP = -(-(N + 1) // 256) * 256
    EP = -(-E // (_NW * 2 * _CH)) * (_NW * 2 * _CH)
    K = EP // (_NW * _CH)

    pad = EP - E
    src = jnp.concatenate([edge_index[0], jnp.full((pad,), N, jnp.int32)])
    dst = jnp.concatenate([edge_index[1], jnp.full((pad,), N, jnp.int32)])
    src3 = src.reshape(_NW, K, _CH)
    dst3 = dst.reshape(_NW, K, _CH)

    PK = 128 // d_h
    NG = NP // PK
    n_convs = n_layers + 1
    n_split = d_out // d_h

    def packed(a):
        return a.reshape(a.shape[:-2] + (NG, PK * d_h))

    def unpacked(a):
        return a.reshape(NP, d_h)

    b_in4 = jnp.tile(b_in.reshape(1, d_h), (1, PK))
    b_hid4 = jnp.tile(b_hid.reshape(n_layers, 1, d_h), (1, 1, PK))
    W_hid4 = jnp.zeros((n_layers, PK * d_h, PK * d_h), jnp.float32)
    for k in range(PK):
        W_hid4 = W_hid4.at[:, k * d_h:(k + 1) * d_h,
                           k * d_h:(k + 1) * d_h].set(W_hid)
    Wor = W_out.reshape(n_convs, d_h, d_out)
    Wout4 = jnp.zeros((n_convs, PK * d_h, PK * d_out), jnp.float32)
    for k in range(PK):
        Wout4 = Wout4.at[:, k * d_h:(k + 1) * d_h,
                         k * d_out:(k + 1) * d_out].set(Wor)
    Wout4 = Wout4.reshape(n_convs * PK * d_h, PK * d_out)

    deg_pass = _build_deg_pass(NP, K)
    edge32 = _build_edge_pass(NP, d_h, K)

    ones_c = jnp.ones((_CH, 8), jnp.float32)
    zeros_c = jnp.zeros((NP // _NS, 8), jnp.float32)
    degS, degD = deg_pass(ones_c, zeros_c, src3, dst3)
    degS = degS.reshape(_NC, NG, PK * 8)
    degD = degD.reshape(_NC, NG, PK * 8)

    NREAL = N // PK
    feats_p = feats.reshape(NREAL, PK * d_in)
    W_in4 = jnp.zeros((PK * d_in, PK * d_h), jnp.float32)
    for k in range(PK):
        W_in4 = W_in4.at[k * d_in:(k + 1) * d_in,
                         k * d_h:(k + 1) * d_h].set(W_in)

    def tc_proj(f_ref, w_ref, z_ref):
        zp = jnp.dot(f_ref[...], w_ref[...], preferred_element_type=jnp.float32)
        tail = jnp.zeros((NG - NREAL, PK * d_h), jnp.float32)
        z_ref[...] = jnp.concatenate([zp, tail], axis=0)

    z_raw = pl.pallas_call(
        tc_proj,
        out_shape=jax.ShapeDtypeStruct((NG, PK * d_h), jnp.float32),
    )(feats_p, W_in4)

    GB = 8
    NGB = NG // GB

    def tc_norms(dS_ref, dD_ref, zr_ref, ns_ref, nd_ref, z_ref):
        dS = dS_ref[0] + dS_ref[1]
        dD = dD_ref[0] + dD_ref[1]

        def spread(d):
            cols = [jnp.broadcast_to(d[:, 8 * k:8 * k + 1], (d.shape[0], d_h))
                    for k in range(PK)]
            return jnp.concatenate(cols, axis=1)

        ns = lax.rsqrt(jnp.maximum(spread(dS), 1.0))
        nd = lax.rsqrt(jnp.maximum(spread(dD), 1.0))
        ns_ref[...] = ns
        nd_ref[...] = nd
        z_ref[...] = zr_ref[...] * ns

    ns_arr, nd_arr, z = pl.pallas_call(
        tc_norms,
        grid=(GB,),
        in_specs=[pl.BlockSpec((_NC, NGB, PK * 8), lambda i: (0, i, 0)),
                  pl.BlockSpec((_NC, NGB, PK * 8), lambda i: (0, i, 0)),
                  pl.BlockSpec((NGB, PK * d_h), lambda i: (i, 0))],
        out_specs=(pl.BlockSpec((NGB, PK * d_h), lambda i: (i, 0)),
                   pl.BlockSpec((NGB, PK * d_h), lambda i: (i, 0)),
                   pl.BlockSpec((NGB, PK * d_h), lambda i: (i, 0))),
        out_shape=(jax.ShapeDtypeStruct((NG, PK * d_h), jnp.float32),
                   jax.ShapeDtypeStruct((NG, PK * d_h), jnp.float32),
                   jax.ShapeDtypeStruct((NG, PK * d_h), jnp.float32)),
    )(degS, degD, z_raw)

    def tc_layer(p_ref, nd_ref, ns_ref, b_ref, w_ref, h_ref, z_ref):
        agg = p_ref[0] + p_ref[1]
        h = jnp.maximum(agg * nd_ref[...] + b_ref[...], 0.0)
        h_ref[...] = h
        z_ref[...] = jnp.dot(h, w_ref[...],
                             preferred_element_type=jnp.float32) * ns_ref[...]

    tc_layer_call = pl.pallas_call(
        tc_layer,
        grid=(GB,),
        in_specs=[pl.BlockSpec((_NC, NGB, PK * d_h), lambda i: (0, i, 0)),
                  pl.BlockSpec((NGB, PK * d_h), lambda i: (i, 0)),
                  pl.BlockSpec((NGB, PK * d_h), lambda i: (i, 0)),
                  pl.BlockSpec((1, PK * d_h), lambda i: (0, 0)),
                  pl.BlockSpec((PK * d_h, PK * d_h), lambda i: (0, 0))],
        out_specs=(pl.BlockSpec((NGB, PK * d_h), lambda i: (i, 0)),
                   pl.BlockSpec((NGB, PK * d_h), lambda i: (i, 0))),
        out_shape=(jax.ShapeDtypeStruct((NG, PK * d_h), jnp.float32),
                   jax.ShapeDtypeStruct((NG, PK * d_h), jnp.float32)),
    )

    hs = []
    for i in range(n_layers):
        part = packed(edge32(unpacked(z), src3, dst3))
        bias = b_in4 if i == 0 else b_hid4[i - 1]
        h, z = tc_layer_call(part, nd_arr, ns_arr, bias, W_hid4[i])
        hs.append(h)
    part_last = packed(edge32(unpacked(z), src3, dst3))

    def tc_jk(p_ref, nd_ref, b_ref, *rest):
        h_refs = rest[:n_layers]
        wout_ref = rest[n_layers]
        out_refs = rest[n_layers + 1:]
        agg = p_ref[0] + p_ref[1]
        h_last = jnp.maximum(agg * nd_ref[...] + b_ref[...], 0.0)
        hcat = jnp.concatenate([r[...] for r in h_refs] + [h_last], axis=1)
        P = jnp.dot(hcat, wout_ref[...], preferred_element_type=jnp.float32)
        for i, o_ref in enumerate(out_refs):
            o_ref[...] = jnp.concatenate(
                [P[:, k * d_out + i * d_h: k * d_out + (i + 1) * d_h]
                 for k in range(PK)], axis=1)

    Ps = pl.pallas_call(
        tc_jk,
        grid=(GB,),
        in_specs=[pl.BlockSpec((_NC, NGB, PK * d_h), lambda i: (0, i, 0)),
                  pl.BlockSpec((NGB, PK * d_h), lambda i: (i, 0)),
                  pl.BlockSpec((1, PK * d_h), lambda i: (0, 0))]
                 + [pl.BlockSpec((NGB, PK * d_h), lambda i: (i, 0))
                    for _ in range(n_layers)]
                 + [pl.BlockSpec((n_convs * PK * d_h, PK * d_out),
                                 lambda i: (0, 0))],
        out_specs=tuple(pl.BlockSpec((NGB, PK * d_h), lambda i: (i, 0))
                        for _ in range(n_split)),
        out_shape=tuple(jax.ShapeDtypeStruct((NG, PK * d_h), jnp.float32)
                        for _ in range(n_split)),
    )(part_last, nd_arr, b_hid4[n_layers - 1], *hs, Wout4)

    partFs = [packed(edge32(unpacked(P_i), src3, dst3)) for P_i in Ps]

    b_out4 = jnp.tile(b_out.reshape(1, d_out), (1, PK))

    def tc_final(*refs):
        p_refs, b_ref, y_ref = refs[:n_split], refs[n_split], refs[n_split + 1]
        fs = [p_ref[0] + p_ref[1] for p_ref in p_refs]
        cols = []
        for k in range(PK):
            for f in fs:
                cols.append(f[:, k * d_h:(k + 1) * d_h])
        y_ref[...] = jnp.concatenate(cols, axis=1) + b_ref[...]

    YB = NG // GB
    y_pk = pl.pallas_call(
        tc_final,
        grid=(GB,),
        in_specs=[pl.BlockSpec((_NC, YB, PK * d_h), lambda i: (0, i, 0))
                  for _ in range(n_split)]
                 + [pl.BlockSpec((1, PK * d_out), lambda i: (0, 0))],
        out_specs=pl.BlockSpec((YB, PK * d_out), lambda i: (i, 0)),
        out_shape=jax.ShapeDtypeStruct((NG, PK * d_out), jnp.float32),
    )(*partFs, b_out4)
    return y_pk.reshape(NP, d_out)[:N]
